# baseline (speedup 1.0000x reference)
import functools

import jax
import jax.numpy as jnp
from jax import lax
from jax.experimental import pallas as pl
from jax.experimental.pallas import tpu as pltpu
from jax.experimental.pallas import tpu_sc as plsc

N = 10000
E = 160000
NT = 16
NC = 2
LN = 128
NP = 10240
NB = 5
RB = NP // NB


def _make_deg_kernel():
  mesh = plsc.VectorSubcoreMesh(core_axis_name="c", subcore_axis_name="s")
  stripe = NP // NT

  @functools.partial(
      pl.kernel,
      out_type=jax.ShapeDtypeStruct((NC, NP, LN), jnp.float32),
      mesh=mesh,
      scratch_types=[
          pltpu.VMEM_SHARED((NP, LN), jnp.float32),
          pltpu.VMEM((_DEG_CH, _DEG_K), jnp.int32),
          pltpu.VMEM((_DEG_K, LN), jnp.float32),
          pltpu.VMEM((40, LN), jnp.float32),
      ],
  )
  def deg_kernel(dst_hbm, ones_hbm, zeros_hbm, out_hbm, cnt, dst_v, ones_v,
                 zeros_v):
    c = lax.axis_index("c")
    s = lax.axis_index("s")
    pltpu.sync_copy(ones_hbm, ones_v)
    pltpu.sync_copy(zeros_hbm, zeros_v)
    for i in range(stripe // 40):
      pltpu.sync_copy(zeros_v, cnt.at[pl.ds(s * stripe + i * 40, 40)])
    pltpu.sync_copy(dst_hbm.at[s, c], dst_v)
    plsc.subcore_barrier()

    def body(j, carry):
      pltpu.sync_copy(ones_v, cnt.at[dst_v.at[j]], add=True)
      return carry

    lax.fori_loop(0, _DEG_CH, body, 0)
    plsc.subcore_barrier()
    pltpu.sync_copy(cnt.at[pl.ds(s * stripe, stripe)],
                    out_hbm.at[c, pl.ds(s * stripe, stripe)])

  return deg_kernel


EPT = E // NT
_AGG_K = 80
_AGG_CH = EPT // _AGG_K
_DEG_K = 100
_DEG_CH = EPT // NC // _DEG_K


def _make_agg_kernel(G):
  mesh = plsc.VectorSubcoreMesh(core_axis_name="c", subcore_axis_name="s")
  stripe = NP // NT
  rounds = G // NC

  @functools.partial(
      pl.kernel,
      out_type=jax.ShapeDtypeStruct((G * NP, LN), jnp.float32),
      mesh=mesh,
      scratch_types=[
          pltpu.VMEM_SHARED((NP, LN), jnp.float32),
          pltpu.VMEM((EPT,), jnp.int32),
          pltpu.VMEM((_AGG_CH, _AGG_K), jnp.int32),
          pltpu.VMEM((2, _AGG_K, LN), jnp.float32),
          pltpu.SemaphoreType.DMA,
          pltpu.SemaphoreType.DMA,
      ],
  )
  def agg_kernel(h2_hbm, src_hbm, dst_hbm, out_hbm, acc, src_v, dst_v, rows_v,
                 sem_a, sem_b):
    c = lax.axis_index("c")
    s = lax.axis_index("s")
    sems = (sem_a, sem_b)
    pltpu.sync_copy(dst_hbm.at[s], dst_v)

    def fire(slot, j):
      pltpu.async_copy(h2_hbm.at[src_v.at[pl.ds(j * _AGG_K, _AGG_K)]],
                       rows_v.at[slot], sems[slot])

    def drain(slot, j):
      pltpu.make_async_copy(h2_hbm.at[src_v.at[pl.ds(j * _AGG_K, _AGG_K)]],
                            rows_v.at[slot], sems[slot]).wait()
      pltpu.sync_copy(rows_v.at[slot], acc.at[dst_v.at[j]], add=True)

    for r in range(rounds):
      g = r * NC + c
      base = g * NP
      pltpu.sync_copy(h2_hbm.at[pl.ds(base + s * stripe, stripe)],
                      acc.at[pl.ds(s * stripe, stripe)])
      pltpu.sync_copy(src_hbm.at[g, s], src_v)
      plsc.subcore_barrier()

      fire(0, 0)
      fire(1, 1)

      def body(jj, carry):
        j = jj * 2
        drain(0, j)
        fire(0, j + 2)
        drain(1, j + 1)
        fire(1, j + 3)
        return carry

      lax.fori_loop(0, _AGG_CH // 2 - 1, body, 0)
      drain(0, _AGG_CH - 3)
      fire(0, _AGG_CH - 1)
      drain(1, _AGG_CH - 2)
      drain(0, _AGG_CH - 1)
      plsc.subcore_barrier()
      pltpu.sync_copy(acc.at[pl.ds(s * stripe, stripe)],
                      out_hbm.at[pl.ds(base + s * stripe, stripe)])

  return agg_kernel


def _mm_body(x_ref, w_ref, out_ref):
  out_ref[...] = jnp.dot(
      x_ref[...], w_ref[...], preferred_element_type=jnp.float32)


def _mm(x, W, G):
  d_in = x.shape[1]
  return pl.pallas_call(
      _mm_body,
      grid=(NB, G),
      in_specs=[
          pl.BlockSpec((RB, d_in), lambda i, g: (i, 0)),
          pl.BlockSpec((d_in, LN), lambda i, g: (0, g)),
      ],
      out_specs=pl.BlockSpec((RB, LN), lambda i, g: (g * NB + i, 0)),
      out_shape=jax.ShapeDtypeStruct((G * NP, LN), jnp.float32),
  )(x, W)


def _scale_body(h_ref, deg_ref, out_ref, dv_ref):
  p = deg_ref[...]
  d = p[0, :, 0:1] + p[1, :, 0:1] + 1.0
  dv = jnp.broadcast_to(lax.rsqrt(d), (RB, LN))
  out_ref[...] = h_ref[...] * dv
  dv_ref[...] = dv


def _scale(h, deg_parts, G):
  return pl.pallas_call(
      _scale_body,
      grid=(NB, G),
      in_specs=[
          pl.BlockSpec((RB, LN), lambda i, g: (g * NB + i, 0)),
          pl.BlockSpec((2, RB, LN), lambda i, g: (0, i, 0)),
      ],
      out_specs=[
          pl.BlockSpec((RB, LN), lambda i, g: (g * NB + i, 0)),
          pl.BlockSpec((RB, LN), lambda i, g: (i, 0)),
      ],
      out_shape=[
          jax.ShapeDtypeStruct((G * NP, LN), jnp.float32),
          jax.ShapeDtypeStruct((NP, LN), jnp.float32),
      ],
  )(h, deg_parts)


def _stats_body(acc_ref, dv_ref, sum_ref, sq_ref):
  i = pl.program_id(1)
  z = acc_ref[...] * dv_ref[...]
  s = jnp.sum(z, axis=0, keepdims=True)[None]
  q = jnp.sum(z * z, axis=0, keepdims=True)[None]

  @pl.when(i == 0)
  def _():
    sum_ref[...] = jnp.zeros_like(sum_ref)
    sq_ref[...] = jnp.zeros_like(sq_ref)

  sum_ref[...] += s
  sq_ref[...] += q


def _stats(acc, dinv, G):
  return pl.pallas_call(
      _stats_body,
      grid=(G, NB),
      in_specs=[
          pl.BlockSpec((RB, LN), lambda g, i: (g * NB + i, 0)),
          pl.BlockSpec((RB, LN), lambda g, i: (i, 0)),
      ],
      out_specs=[
          pl.BlockSpec((1, 1, LN), lambda g, i: (g, 0, 0)),
          pl.BlockSpec((1, 1, LN), lambda g, i: (g, 0, 0)),
      ],
      out_shape=[
          jax.ShapeDtypeStruct((G, 1, LN), jnp.float32),
          jax.ShapeDtypeStruct((G, 1, LN), jnp.float32),
      ],
  )(acc, dinv)


_EPS = 1e-5


def _bn_relu(acc_ref, dv_ref, sum_ref, sq_ref, gam_ref, bet_ref, row_block):
  z = acc_ref[...] * dv_ref[...]
  mean = sum_ref[0, 0, :] * (1.0 / N)
  var = sq_ref[0, 0, :] * (1.0 / N) - mean * mean
  r = (z - mean) * (gam_ref[0, 0, :] * lax.rsqrt(var + _EPS)) + bet_ref[0, 0, :]
  r = jnp.maximum(r, 0.0)
  rows = lax.broadcasted_iota(jnp.int32, (RB, 1), 0) + row_block * RB
  return jnp.where(rows < N, r, 0.0)


def _layer2_body(acc_ref, dv_ref, sum_ref, sq_ref, gam_ref, bet_ref,
                 w_ref, out_ref, *, g_in):
  g1 = pl.program_id(2)
  r = _bn_relu(acc_ref, dv_ref, sum_ref, sq_ref, gam_ref, bet_ref,
               pl.program_id(1))
  contrib = jnp.dot(r, w_ref[...], preferred_element_type=jnp.float32)

  @pl.when(g1 == 0)
  def _():
    out_ref[...] = contrib

  @pl.when(jnp.logical_and(g1 > 0, g1 < g_in - 1))
  def _():
    out_ref[...] += contrib

  @pl.when(g1 == g_in - 1)
  def _():
    out_ref[...] = (out_ref[...] + contrib) * dv_ref[...]


def _layer2(acc, dinv, sums, sq, gamma, beta, W, g_in, g_out):
  return pl.pallas_call(
      functools.partial(_layer2_body, g_in=g_in),
      grid=(g_out, NB, g_in),
      in_specs=[
          pl.BlockSpec((RB, LN), lambda g2, i, g1: (g1 * NB + i, 0)),
          pl.BlockSpec((RB, LN), lambda g2, i, g1: (i, 0)),
          pl.BlockSpec((1, 1, LN), lambda g2, i, g1: (g1, 0, 0)),
          pl.BlockSpec((1, 1, LN), lambda g2, i, g1: (g1, 0, 0)),
          pl.BlockSpec((1, 1, LN), lambda g2, i, g1: (g1, 0, 0)),
          pl.BlockSpec((1, 1, LN), lambda g2, i, g1: (g1, 0, 0)),
          pl.BlockSpec((LN, LN), lambda g2, i, g1: (g1, g2)),
      ],
      out_specs=pl.BlockSpec((RB, LN), lambda g2, i, g1: (g2 * NB + i, 0)),
      out_shape=jax.ShapeDtypeStruct((g_out * NP, LN), jnp.float32),
  )(acc, dinv, sums, sq, gamma, beta, W)


def _head_body(acc_ref, dv_ref, sum_ref, sq_ref, gam_ref, bet_ref,
               w_ref, fcb_ref, out_ref, *, g_in):
  g2 = pl.program_id(1)
  r = _bn_relu(acc_ref, dv_ref, sum_ref, sq_ref, gam_ref, bet_ref,
               pl.program_id(0))
  w = w_ref[0, :, 0]
  contrib = jnp.sum(r * w[None, :], axis=1, keepdims=True)

  @pl.when(g2 == 0)
  def _():
    out_ref[...] = contrib

  @pl.when(jnp.logical_and(g2 > 0, g2 < g_in - 1))
  def _():
    out_ref[...] += contrib

  @pl.when(g2 == g_in - 1)
  def _():
    out_ref[...] = out_ref[...] + contrib + fcb_ref[0, 0]


def _head(acc, dinv, sums, sq, gamma, beta, fcW, fcb, g_in):
  return pl.pallas_call(
      functools.partial(_head_body, g_in=g_in),
      grid=(NB, g_in),
      in_specs=[
          pl.BlockSpec((RB, LN), lambda i, g2: (g2 * NB + i, 0)),
          pl.BlockSpec((RB, LN), lambda i, g2: (i, 0)),
          pl.BlockSpec((1, 1, LN), lambda i, g2: (g2, 0, 0)),
          pl.BlockSpec((1, 1, LN), lambda i, g2: (g2, 0, 0)),
          pl.BlockSpec((1, 1, LN), lambda i, g2: (g2, 0, 0)),
          pl.BlockSpec((1, 1, LN), lambda i, g2: (g2, 0, 0)),
          pl.BlockSpec((1, LN, 1), lambda i, g2: (g2, 0, 0)),
          pl.BlockSpec((1, 1), lambda i, g2: (0, 0)),
      ],
      out_specs=pl.BlockSpec((RB, 1), lambda i, g2: (i, 0)),
      out_shape=jax.ShapeDtypeStruct((NP, 1), jnp.float32),
  )(acc, dinv, sums, sq, gamma, beta, fcW, fcb)


_deg_call = _make_deg_kernel()
_agg4_call = _make_agg_kernel(4)
_agg2_call = _make_agg_kernel(2)


def kernel(x, edge_index, W1, b1, gamma1, beta1, W2, b2, gamma2, beta2, fcW,
           fcb):
  src = edge_index[0]
  dst = edge_index[1]

  dst_agg = dst.reshape(NT, _AGG_CH, _AGG_K)
  dst_deg = dst.reshape(NT, NC, _DEG_CH, _DEG_K)
  ones_rows = jnp.ones((_DEG_K, LN), jnp.float32)
  zeros_rows = jnp.zeros((40, LN), jnp.float32)
  deg_parts = _deg_call(dst_deg, ones_rows, zeros_rows)

  goff4 = (jnp.arange(4, dtype=jnp.int32) * NP)[:, None]
  src4 = (src[None, :] + goff4).reshape(4, NT, EPT)
  src2 = (src[None, :] + goff4[:2]).reshape(2, NT, EPT)

  x_pad = jnp.pad(x, ((0, NP - N), (0, 0)))
  h1 = _mm(x_pad, W1, 4)
  h2, dinv = _scale(h1, deg_parts, 4)
  agg1 = _agg4_call(h2, src4, dst_agg)

  s1, q1 = _stats(agg1, dinv, 4)
  h2b = _layer2(agg1, dinv, s1, q1, gamma1.reshape(4, 1, LN),
                beta1.reshape(4, 1, LN), W2, 4, 2)

  agg2 = _agg2_call(h2b, src2, dst_agg)

  s2, q2 = _stats(agg2, dinv, 2)
  out = _head(agg2, dinv, s2, q2, gamma2.reshape(2, 1, LN),
              beta2.reshape(2, 1, LN), fcW.reshape(2, LN, 1),
              fcb.reshape(1, 1), 2)
  return out[:N]


# speedup vs baseline: 1.0045x; 1.0045x over previous
import functools

import jax
import jax.numpy as jnp
from jax import lax
from jax.experimental import pallas as pl
from jax.experimental.pallas import tpu as pltpu
from jax.experimental.pallas import tpu_sc as plsc

N = 10000
E = 160000
NT = 16
NC = 2
LN = 128
NP = 10240
NB = 5
RB = NP // NB


def _make_deg_kernel():
  mesh = plsc.VectorSubcoreMesh(core_axis_name="c", subcore_axis_name="s")
  stripe = NP // NT

  @functools.partial(
      pl.kernel,
      out_type=jax.ShapeDtypeStruct((NC, NP, LN), jnp.float32),
      mesh=mesh,
      scratch_types=[
          pltpu.VMEM_SHARED((NP, LN), jnp.float32),
          pltpu.VMEM((_DEG_CH, _DEG_K), jnp.int32),
          pltpu.VMEM((_DEG_K, LN), jnp.float32),
          pltpu.VMEM((40, LN), jnp.float32),
      ],
  )
  def deg_kernel(dst_hbm, ones_hbm, zeros_hbm, out_hbm, cnt, dst_v, ones_v,
                 zeros_v):
    c = lax.axis_index("c")
    s = lax.axis_index("s")
    pltpu.sync_copy(ones_hbm, ones_v)
    pltpu.sync_copy(zeros_hbm, zeros_v)
    for i in range(stripe // 40):
      pltpu.sync_copy(zeros_v, cnt.at[pl.ds(s * stripe + i * 40, 40)])
    pltpu.sync_copy(dst_hbm.at[s, c], dst_v)
    plsc.subcore_barrier()

    def body(j, carry):
      pltpu.sync_copy(ones_v, cnt.at[dst_v.at[j]], add=True)
      return carry

    lax.fori_loop(0, _DEG_CH, body, 0)
    plsc.subcore_barrier()
    pltpu.sync_copy(cnt.at[pl.ds(s * stripe, stripe)],
                    out_hbm.at[c, pl.ds(s * stripe, stripe)])

  return deg_kernel


EPT = E // NT
_AGG_K = 80
_AGG_CH = EPT // _AGG_K
_DEG_K = 100
_DEG_CH = EPT // NC // _DEG_K


def _make_agg_kernel(G):
  mesh = plsc.VectorSubcoreMesh(core_axis_name="c", subcore_axis_name="s")
  stripe = NP // NT
  rounds = G // NC

  @functools.partial(
      pl.kernel,
      out_type=jax.ShapeDtypeStruct((G * NP, LN), jnp.float32),
      mesh=mesh,
      scratch_types=[
          pltpu.VMEM_SHARED((NP, LN), jnp.float32),
          pltpu.VMEM((EPT,), jnp.int32),
          pltpu.VMEM((_AGG_CH, _AGG_K), jnp.int32),
          pltpu.VMEM((2, _AGG_K, LN), jnp.float32),
          pltpu.SemaphoreType.DMA,
          pltpu.SemaphoreType.DMA,
      ],
  )
  def agg_kernel(h2_hbm, src_hbm, dst_hbm, out_hbm, acc, src_v, dst_v, rows_v,
                 sem_a, sem_b):
    c = lax.axis_index("c")
    s = lax.axis_index("s")
    sems = (sem_a, sem_b)
    pltpu.sync_copy(dst_hbm.at[s], dst_v)

    def fire(slot, j):
      pltpu.async_copy(h2_hbm.at[src_v.at[pl.ds(j * _AGG_K, _AGG_K)]],
                       rows_v.at[slot], sems[slot])

    def drain(slot, j):
      pltpu.make_async_copy(h2_hbm.at[src_v.at[pl.ds(j * _AGG_K, _AGG_K)]],
                            rows_v.at[slot], sems[slot]).wait()
      pltpu.sync_copy(rows_v.at[slot], acc.at[dst_v.at[j]], add=True)

    for r in range(rounds):
      g = r * NC + c
      base = g * NP
      pltpu.sync_copy(h2_hbm.at[pl.ds(base + s * stripe, stripe)],
                      acc.at[pl.ds(s * stripe, stripe)])
      pltpu.sync_copy(src_hbm.at[g, s], src_v)
      plsc.subcore_barrier()

      fire(0, 0)
      fire(1, 1)

      def body(jj, carry):
        j = jj * 2
        drain(0, j)
        fire(0, j + 2)
        drain(1, j + 1)
        fire(1, j + 3)
        return carry

      lax.fori_loop(0, _AGG_CH // 2 - 1, body, 0)
      drain(0, _AGG_CH - 3)
      fire(0, _AGG_CH - 1)
      drain(1, _AGG_CH - 2)
      drain(0, _AGG_CH - 1)
      plsc.subcore_barrier()
      pltpu.sync_copy(acc.at[pl.ds(s * stripe, stripe)],
                      out_hbm.at[pl.ds(base + s * stripe, stripe)])

  return agg_kernel


def _h2_body(x_ref, w_ref, deg_ref, out_ref, dv_ref):
  p = deg_ref[...]
  d = p[0, :, 0:1] + p[1, :, 0:1] + 1.0
  dv = jnp.broadcast_to(lax.rsqrt(d), (RB, LN))
  out_ref[...] = jnp.dot(
      x_ref[...], w_ref[...], preferred_element_type=jnp.float32
  ) * dv
  dv_ref[...] = dv


def _h2(x, W, deg_parts, G):
  d_in = x.shape[1]
  return pl.pallas_call(
      _h2_body,
      grid=(NB, G),
      in_specs=[
          pl.BlockSpec((RB, d_in), lambda i, g: (i, 0)),
          pl.BlockSpec((d_in, LN), lambda i, g: (0, g)),
          pl.BlockSpec((2, RB, LN), lambda i, g: (0, i, 0)),
      ],
      out_specs=[
          pl.BlockSpec((RB, LN), lambda i, g: (g * NB + i, 0)),
          pl.BlockSpec((RB, LN), lambda i, g: (i, 0)),
      ],
      out_shape=[
          jax.ShapeDtypeStruct((G * NP, LN), jnp.float32),
          jax.ShapeDtypeStruct((NP, LN), jnp.float32),
      ],
  )(x, W, deg_parts)


def _stats_body(acc_ref, dv_ref, sum_ref, sq_ref):
  i = pl.program_id(1)
  z = acc_ref[...] * dv_ref[...]
  s = jnp.sum(z, axis=0, keepdims=True)[None]
  q = jnp.sum(z * z, axis=0, keepdims=True)[None]

  @pl.when(i == 0)
  def _():
    sum_ref[...] = jnp.zeros_like(sum_ref)
    sq_ref[...] = jnp.zeros_like(sq_ref)

  sum_ref[...] += s
  sq_ref[...] += q


def _stats(acc, dinv, G):
  return pl.pallas_call(
      _stats_body,
      grid=(G, NB),
      in_specs=[
          pl.BlockSpec((RB, LN), lambda g, i: (g * NB + i, 0)),
          pl.BlockSpec((RB, LN), lambda g, i: (i, 0)),
      ],
      out_specs=[
          pl.BlockSpec((1, 1, LN), lambda g, i: (g, 0, 0)),
          pl.BlockSpec((1, 1, LN), lambda g, i: (g, 0, 0)),
      ],
      out_shape=[
          jax.ShapeDtypeStruct((G, 1, LN), jnp.float32),
          jax.ShapeDtypeStruct((G, 1, LN), jnp.float32),
      ],
  )(acc, dinv)


_EPS = 1e-5


def _bn_relu(acc_ref, dv_ref, sum_ref, sq_ref, gam_ref, bet_ref, row_block):
  z = acc_ref[...] * dv_ref[...]
  mean = sum_ref[0, 0, :] * (1.0 / N)
  var = sq_ref[0, 0, :] * (1.0 / N) - mean * mean
  r = (z - mean) * (gam_ref[0, 0, :] * lax.rsqrt(var + _EPS)) + bet_ref[0, 0, :]
  r = jnp.maximum(r, 0.0)
  rows = lax.broadcasted_iota(jnp.int32, (RB, 1), 0) + row_block * RB
  return jnp.where(rows < N, r, 0.0)


def _layer2_body(acc_ref, dv_ref, sum_ref, sq_ref, gam_ref, bet_ref,
                 w_ref, out_ref, *, g_in):
  g1 = pl.program_id(2)
  r = _bn_relu(acc_ref, dv_ref, sum_ref, sq_ref, gam_ref, bet_ref,
               pl.program_id(1))
  contrib = jnp.dot(r, w_ref[...], preferred_element_type=jnp.float32)

  @pl.when(g1 == 0)
  def _():
    out_ref[...] = contrib

  @pl.when(jnp.logical_and(g1 > 0, g1 < g_in - 1))
  def _():
    out_ref[...] += contrib

  @pl.when(g1 == g_in - 1)
  def _():
    out_ref[...] = (out_ref[...] + contrib) * dv_ref[...]


def _layer2(acc, dinv, sums, sq, gamma, beta, W, g_in, g_out):
  return pl.pallas_call(
      functools.partial(_layer2_body, g_in=g_in),
      grid=(g_out, NB, g_in),
      in_specs=[
          pl.BlockSpec((RB, LN), lambda g2, i, g1: (g1 * NB + i, 0)),
          pl.BlockSpec((RB, LN), lambda g2, i, g1: (i, 0)),
          pl.BlockSpec((1, 1, LN), lambda g2, i, g1: (g1, 0, 0)),
          pl.BlockSpec((1, 1, LN), lambda g2, i, g1: (g1, 0, 0)),
          pl.BlockSpec((1, 1, LN), lambda g2, i, g1: (g1, 0, 0)),
          pl.BlockSpec((1, 1, LN), lambda g2, i, g1: (g1, 0, 0)),
          pl.BlockSpec((LN, LN), lambda g2, i, g1: (g1, g2)),
      ],
      out_specs=pl.BlockSpec((RB, LN), lambda g2, i, g1: (g2 * NB + i, 0)),
      out_shape=jax.ShapeDtypeStruct((g_out * NP, LN), jnp.float32),
  )(acc, dinv, sums, sq, gamma, beta, W)


def _head_body(acc_ref, dv_ref, sum_ref, sq_ref, gam_ref, bet_ref,
               w_ref, fcb_ref, out_ref, *, g_in):
  g2 = pl.program_id(1)
  r = _bn_relu(acc_ref, dv_ref, sum_ref, sq_ref, gam_ref, bet_ref,
               pl.program_id(0))
  w = w_ref[0, :, 0]
  contrib = jnp.sum(r * w[None, :], axis=1, keepdims=True)

  @pl.when(g2 == 0)
  def _():
    out_ref[...] = contrib

  @pl.when(jnp.logical_and(g2 > 0, g2 < g_in - 1))
  def _():
    out_ref[...] += contrib

  @pl.when(g2 == g_in - 1)
  def _():
    out_ref[...] = out_ref[...] + contrib + fcb_ref[0, 0]


def _head(acc, dinv, sums, sq, gamma, beta, fcW, fcb, g_in):
  return pl.pallas_call(
      functools.partial(_head_body, g_in=g_in),
      grid=(NB, g_in),
      in_specs=[
          pl.BlockSpec((RB, LN), lambda i, g2: (g2 * NB + i, 0)),
          pl.BlockSpec((RB, LN), lambda i, g2: (i, 0)),
          pl.BlockSpec((1, 1, LN), lambda i, g2: (g2, 0, 0)),
          pl.BlockSpec((1, 1, LN), lambda i, g2: (g2, 0, 0)),
          pl.BlockSpec((1, 1, LN), lambda i, g2: (g2, 0, 0)),
          pl.BlockSpec((1, 1, LN), lambda i, g2: (g2, 0, 0)),
          pl.BlockSpec((1, LN, 1), lambda i, g2: (g2, 0, 0)),
          pl.BlockSpec((1, 1), lambda i, g2: (0, 0)),
      ],
      out_specs=pl.BlockSpec((RB, 1), lambda i, g2: (i, 0)),
      out_shape=jax.ShapeDtypeStruct((NP, 1), jnp.float32),
  )(acc, dinv, sums, sq, gamma, beta, fcW, fcb)


_deg_call = _make_deg_kernel()
_agg4_call = _make_agg_kernel(4)
_agg2_call = _make_agg_kernel(2)


def kernel(x, edge_index, W1, b1, gamma1, beta1, W2, b2, gamma2, beta2, fcW,
           fcb):
  src = edge_index[0]
  dst = edge_index[1]

  dst_agg = dst.reshape(NT, _AGG_CH, _AGG_K)
  dst_deg = dst.reshape(NT, NC, _DEG_CH, _DEG_K)
  ones_rows = jnp.ones((_DEG_K, LN), jnp.float32)
  zeros_rows = jnp.zeros((40, LN), jnp.float32)
  deg_parts = _deg_call(dst_deg, ones_rows, zeros_rows)

  goff4 = (jnp.arange(4, dtype=jnp.int32) * NP)[:, None]
  src4 = (src[None, :] + goff4).reshape(4, NT, EPT)
  src2 = (src[None, :] + goff4[:2]).reshape(2, NT, EPT)

  x_pad = jnp.pad(x, ((0, NP - N), (0, 0)))
  h2, dinv = _h2(x_pad, W1, deg_parts, 4)
  agg1 = _agg4_call(h2, src4, dst_agg)

  s1, q1 = _stats(agg1, dinv, 4)
  h2b = _layer2(agg1, dinv, s1, q1, gamma1.reshape(4, 1, LN),
                beta1.reshape(4, 1, LN), W2, 4, 2)

  agg2 = _agg2_call(h2b, src2, dst_agg)

  s2, q2 = _stats(agg2, dinv, 2)
  out = _head(agg2, dinv, s2, q2, gamma2.reshape(2, 1, LN),
              beta2.reshape(2, 1, LN), fcW.reshape(2, LN, 1),
              fcb.reshape(1, 1), 2)
  return out[:N]
